# Initial kernel scaffold; baseline (speedup 1.0000x reference)
import functools
import jax
import jax.numpy as jnp
from jax import lax
from jax.experimental import pallas as pl
from jax.experimental.pallas import tpu as pltpu
from jax.experimental.pallas import tpu_sc as plsc


_BN = 500


def _rel_tables_body(x_ref, w_ref, o_ref):
    o_ref[0] = jax.lax.dot_general(
        x_ref[...], w_ref[0], (((1,), (0,)), ((), ())),
        precision=jax.lax.Precision.HIGHEST,
        preferred_element_type=jnp.float32)


def _rel_tables(x, W):
    n, din = x.shape
    r, _, dout = W.shape
    nb = n // _BN
    return pl.pallas_call(
        _rel_tables_body,
        grid=(r, nb),
        in_specs=[
            pl.BlockSpec((_BN, din), lambda ri, i: (i, 0)),
            pl.BlockSpec((1, din, dout), lambda ri, i: (ri, 0, 0)),
        ],
        out_specs=pl.BlockSpec((1, _BN, dout), lambda ri, i: (ri, i, 0)),
        out_shape=jax.ShapeDtypeStruct((r, n, dout), jnp.float32),
    )(x, W)


def _root_body(x_ref, w_ref, b_ref, o_ref):
    o_ref[...] = jax.lax.dot_general(
        x_ref[...], w_ref[...], (((1,), (0,)), ((), ())),
        precision=jax.lax.Precision.HIGHEST,
        preferred_element_type=jnp.float32) + b_ref[...]


def _root_transform(x, root, b):
    n, din = x.shape
    dout = root.shape[1]
    nb = n // _BN
    return pl.pallas_call(
        _root_body,
        grid=(nb,),
        in_specs=[
            pl.BlockSpec((_BN, din), lambda i: (i, 0)),
            pl.BlockSpec((din, dout), lambda i: (0, 0)),
            pl.BlockSpec((dout,), lambda i: (0,)),
        ],
        out_specs=pl.BlockSpec((_BN, dout), lambda i: (i, 0)),
        out_shape=jax.ShapeDtypeStruct((n, dout), jnp.float32),
    )(x, root, b)


def _norm_body(d_ref, o_ref):
    deg = d_ref[0] + d_ref[1]
    o_ref[...] = 1.0 / jnp.maximum(deg, 1.0)


def _norm_from_deg(deg2):
    _, rn, w = deg2.shape
    blk = 2000
    return pl.pallas_call(
        _norm_body,
        grid=(rn // blk,),
        in_specs=[pl.BlockSpec((2, blk, w), lambda i: (0, i, 0))],
        out_specs=pl.BlockSpec((blk, w), lambda i: (i, 0)),
        out_shape=jax.ShapeDtypeStruct((rn, w), jnp.float32),
    )(deg2)


def _combine_body(p_ref, xr_ref, o_ref, *, relu):
    v = p_ref[0] + p_ref[1] + xr_ref[...]
    if relu:
        v = jnp.maximum(v, 0.0)
    o_ref[...] = v


def _combine(p, xr, relu):
    n, d = xr.shape
    nb = n // _BN
    return pl.pallas_call(
        functools.partial(_combine_body, relu=relu),
        grid=(nb,),
        in_specs=[
            pl.BlockSpec((2, _BN, d), lambda i: (0, i, 0)),
            pl.BlockSpec((_BN, d), lambda i: (i, 0)),
        ],
        out_specs=pl.BlockSpec((_BN, d), lambda i: (i, 0)),
        out_shape=jax.ShapeDtypeStruct((n, d), jnp.float32),
    )(p, xr)


def _deg_stub(edge_type, dst, n, r):
    rn = r * n
    key = edge_type * n + dst
    half = key.shape[0] // 2
    d0 = jnp.zeros((rn,), jnp.float32).at[key[:half]].add(1.0)
    d1 = jnp.zeros((rn,), jnp.float32).at[key[half:]].add(1.0)
    deg2 = jnp.stack([d0, d1])
    return jnp.broadcast_to(deg2[:, :, None], (2, rn, 16))


def _agg_stub(table, norm, edge_type, src, dst, n):
    d = table.shape[1]
    key_src = edge_type * n + src
    key_dst = edge_type * n + dst
    msg = table[key_src] * norm[key_dst, 0][:, None]
    half = src.shape[0] // 2
    p0 = jnp.zeros((n, d), jnp.float32).at[dst[:half]].add(msg[:half])
    p1 = jnp.zeros((n, d), jnp.float32).at[dst[half:]].add(msg[half:])
    return jnp.stack([p0, p1])


def kernel(x, edge_index, edge_type, W1, root1, b1, W2, root2, b2):
    n, din = x.shape
    r = W1.shape[0]
    src = edge_index[0]
    dst = edge_index[1]

    deg2 = _deg_stub(edge_type, dst, n, r)
    norm = _norm_from_deg(deg2)

    t1 = _rel_tables(x, W1).reshape(r * n, -1)
    xr1 = _root_transform(x, root1, b1)
    p1 = _agg_stub(t1, norm, edge_type, src, dst, n)
    h = _combine(p1, xr1, relu=True)

    t2 = _rel_tables(h, W2).reshape(r * n, -1)
    xr2 = _root_transform(h, root2, b2)
    p2 = _agg_stub(t2, norm, edge_type, src, dst, n)
    return _combine(p2, xr2, relu=False)


# speedup vs baseline: 3.6880x; 3.6880x over previous
import functools
import jax
import jax.numpy as jnp
from jax import lax
from jax.experimental import pallas as pl
from jax.experimental.pallas import tpu as pltpu
from jax.experimental.pallas import tpu_sc as plsc


_BN = 400


def _rel_tables_body(x_ref, w_ref, o_ref):
    o_ref[0] = jax.lax.dot_general(
        x_ref[...], w_ref[0], (((1,), (0,)), ((), ())),
        precision=jax.lax.Precision.HIGHEST,
        preferred_element_type=jnp.float32)


def _rel_tables(x, W):
    n, din = x.shape
    r, _, dout = W.shape
    nb = n // _BN
    return pl.pallas_call(
        _rel_tables_body,
        grid=(r, nb),
        in_specs=[
            pl.BlockSpec((_BN, din), lambda ri, i: (i, 0)),
            pl.BlockSpec((1, din, dout), lambda ri, i: (ri, 0, 0)),
        ],
        out_specs=pl.BlockSpec((1, _BN, dout), lambda ri, i: (ri, i, 0)),
        out_shape=jax.ShapeDtypeStruct((r, n, dout), jnp.float32),
    )(x, W)


def _root_body(x_ref, w_ref, b_ref, o_ref):
    o_ref[...] = jax.lax.dot_general(
        x_ref[...], w_ref[...], (((1,), (0,)), ((), ())),
        precision=jax.lax.Precision.HIGHEST,
        preferred_element_type=jnp.float32) + b_ref[...]


def _root_transform(x, root, b):
    n, din = x.shape
    dout = root.shape[1]
    nb = n // _BN
    return pl.pallas_call(
        _root_body,
        grid=(nb,),
        in_specs=[
            pl.BlockSpec((_BN, din), lambda i: (i, 0)),
            pl.BlockSpec((din, dout), lambda i: (0, 0)),
            pl.BlockSpec((dout,), lambda i: (0,)),
        ],
        out_specs=pl.BlockSpec((_BN, dout), lambda i: (i, 0)),
        out_shape=jax.ShapeDtypeStruct((n, dout), jnp.float32),
    )(x, root, b)


def _norm_body(d_ref, o_ref):
    deg = d_ref[0] + d_ref[1]
    o_ref[...] = 1.0 / jnp.maximum(deg, 1.0)


def _norm_from_deg(deg2):
    _, rn, w = deg2.shape
    blk = 2000
    return pl.pallas_call(
        _norm_body,
        grid=(rn // blk,),
        in_specs=[pl.BlockSpec((2, blk, w), lambda i: (0, i, 0))],
        out_specs=pl.BlockSpec((blk, w), lambda i: (i, 0)),
        out_shape=jax.ShapeDtypeStruct((rn, w), jnp.float32),
    )(deg2)


def _combine_body(p_ref, xr_ref, o_ref, *, relu):
    v = p_ref[0] + p_ref[1] + xr_ref[...]
    if relu:
        v = jnp.maximum(v, 0.0)
    o_ref[...] = v


def _combine(p, xr, relu):
    n, d = xr.shape
    nb = n // _BN
    return pl.pallas_call(
        functools.partial(_combine_body, relu=relu),
        grid=(nb,),
        in_specs=[
            pl.BlockSpec((2, _BN, d), lambda i: (0, i, 0)),
            pl.BlockSpec((_BN, d), lambda i: (i, 0)),
        ],
        out_specs=pl.BlockSpec((_BN, d), lambda i: (i, 0)),
        out_shape=jax.ShapeDtypeStruct((n, d), jnp.float32),
    )(p, xr)


def _deg_stub(edge_type, dst, n, r):
    rn = r * n
    key = edge_type * n + dst
    half = key.shape[0] // 2
    d0 = jnp.zeros((rn,), jnp.float32).at[key[:half]].add(1.0)
    d1 = jnp.zeros((rn,), jnp.float32).at[key[half:]].add(1.0)
    deg2 = jnp.stack([d0, d1])
    return jnp.broadcast_to(deg2[:, :, None], (2, rn, 16))


def _agg_stub(table, norm, edge_type, src, dst, n):
    d = table.shape[1]
    key_src = edge_type * n + src
    key_dst = edge_type * n + dst
    msg = table[key_src] * norm[key_dst, 0][:, None]
    half = src.shape[0] // 2
    p0 = jnp.zeros((n, d), jnp.float32).at[dst[:half]].add(msg[:half])
    p1 = jnp.zeros((n, d), jnp.float32).at[dst[half:]].add(msg[half:])
    return jnp.stack([p0, p1])


def kernel(x, edge_index, edge_type, W1, root1, b1, W2, root2, b2):
    n, din = x.shape
    r = W1.shape[0]
    src = edge_index[0]
    dst = edge_index[1]

    deg2 = _deg_stub(edge_type, dst, n, r)
    norm = _norm_from_deg(deg2)

    t1 = _rel_tables(x, W1).reshape(r * n, -1)
    xr1 = _root_transform(x, root1, b1)
    p1 = _agg_stub(t1, norm, edge_type, src, dst, n)
    h = _combine(p1, xr1, relu=True)

    t2 = _rel_tables(h, W2).reshape(r * n, -1)
    xr2 = _root_transform(h, root2, b2)
    p2 = _agg_stub(t2, norm, edge_type, src, dst, n)
    return _combine(p2, xr2, relu=False)


# speedup vs baseline: 4.5809x; 1.2421x over previous
import functools
import jax
import jax.numpy as jnp
from jax import lax
from jax.experimental import pallas as pl
from jax.experimental.pallas import tpu as pltpu
from jax.experimental.pallas import tpu_sc as plsc


_BN = 400


def _rel_tables_body(x_ref, w_ref, o_ref):
    o_ref[0] = jax.lax.dot_general(
        x_ref[...], w_ref[0], (((1,), (0,)), ((), ())),
        precision=jax.lax.Precision.HIGHEST,
        preferred_element_type=jnp.float32)


def _rel_tables(x, W):
    n, din = x.shape
    r, _, dout = W.shape
    nb = n // _BN
    return pl.pallas_call(
        _rel_tables_body,
        grid=(r, nb),
        in_specs=[
            pl.BlockSpec((_BN, din), lambda ri, i: (i, 0)),
            pl.BlockSpec((1, din, dout), lambda ri, i: (ri, 0, 0)),
        ],
        out_specs=pl.BlockSpec((1, _BN, dout), lambda ri, i: (ri, i, 0)),
        out_shape=jax.ShapeDtypeStruct((r, n, dout), jnp.float32),
    )(x, W)


def _root_body(x_ref, w_ref, b_ref, o_ref):
    o_ref[...] = jax.lax.dot_general(
        x_ref[...], w_ref[...], (((1,), (0,)), ((), ())),
        precision=jax.lax.Precision.HIGHEST,
        preferred_element_type=jnp.float32) + b_ref[...]


def _root_transform(x, root, b):
    n, din = x.shape
    dout = root.shape[1]
    nb = n // _BN
    return pl.pallas_call(
        _root_body,
        grid=(nb,),
        in_specs=[
            pl.BlockSpec((_BN, din), lambda i: (i, 0)),
            pl.BlockSpec((din, dout), lambda i: (0, 0)),
            pl.BlockSpec((dout,), lambda i: (0,)),
        ],
        out_specs=pl.BlockSpec((_BN, dout), lambda i: (i, 0)),
        out_shape=jax.ShapeDtypeStruct((n, dout), jnp.float32),
    )(x, root, b)


def _norm_body(d_ref, o_ref, *, r):
    deg = d_ref[0] + d_ref[1]
    cols = []
    for ri in range(r):
        col = deg[:, 16 * ri:16 * ri + 1]
        cols.append(jnp.broadcast_to(
            1.0 / jnp.maximum(col, 1.0), (deg.shape[0], 128)))
    blk = jnp.concatenate(cols, axis=1)
    o_ref[...] = blk.reshape(-1, 128)


def _norm128_from_deg(deg2, n, r):
    nb = n // _BN
    return pl.pallas_call(
        functools.partial(_norm_body, r=r),
        grid=(nb,),
        in_specs=[pl.BlockSpec((2, _BN, 128), lambda i: (0, i, 0))],
        out_specs=pl.BlockSpec((_BN * r, 128), lambda i: (i, 0)),
        out_shape=jax.ShapeDtypeStruct((n * r, 128), jnp.float32),
    )(deg2)


def _combine_body(p_ref, xr_ref, o_ref, *, relu):
    v = p_ref[0] + p_ref[1] + xr_ref[...]
    if relu:
        v = jnp.maximum(v, 0.0)
    o_ref[...] = v


def _combine(p, xr, relu):
    n, d = xr.shape
    nb = n // _BN
    return pl.pallas_call(
        functools.partial(_combine_body, relu=relu),
        grid=(nb,),
        in_specs=[
            pl.BlockSpec((2, _BN, d), lambda i: (0, i, 0)),
            pl.BlockSpec((_BN, d), lambda i: (i, 0)),
        ],
        out_specs=pl.BlockSpec((_BN, d), lambda i: (i, 0)),
        out_shape=jax.ShapeDtypeStruct((n, d), jnp.float32),
    )(p, xr)


_K = 80
_ZB = 40

_MESH = dict(core_axis_name="c", subcore_axis_name="s")


def _onehot_body(o_ref):
    grp = jax.lax.broadcasted_iota(jnp.int32, o_ref.shape, 1) // 16
    row = jax.lax.broadcasted_iota(jnp.int32, o_ref.shape, 0)
    o_ref[...] = jnp.where(grp == row, 1.0, 0.0)


def _onehot_table(r):
    return pl.pallas_call(
        _onehot_body,
        out_shape=jax.ShapeDtypeStruct((r, 128), jnp.float32),
    )()


def _sc_deg(edge_type, dst, oh_tbl, n, r):
    e = edge_type.shape[0]
    nc, ns = 2, 16
    ep = e // (nc * ns)
    nch = ep // _K
    zt = 10
    rows_t = n // zt
    mesh = plsc.VectorSubcoreMesh(**_MESH)

    @functools.partial(
        pl.kernel,
        out_type=jax.ShapeDtypeStruct((nc, n, 128), jnp.float32),
        mesh=mesh,
        scratch_types=[
            pltpu.VMEM_SHARED((n, 128), jnp.float32),
            pltpu.VMEM((_K,), jnp.int32),
            pltpu.VMEM((_K,), jnp.int32),
            pltpu.VMEM((_K, 128), jnp.float32),
            pltpu.VMEM((_ZB, 128), jnp.float32),
            pltpu.SemaphoreType.DMA,
        ],
    )
    def deg_kernel(et_hbm, dst_hbm, oh_hbm, out_hbm, acc_sh, tv, dv, val_v,
                   stage_v, sem1):
        cid = lax.axis_index("c")
        sid = lax.axis_index("s")
        zero16 = jnp.zeros((16,), jnp.float32)
        for j in range(_ZB):
            for b_ in range(8):
                stage_v[j, pl.ds(b_ * 16, 16)] = zero16
        row0 = sid * rows_t

        @pl.when(sid < zt)
        def _zero():
            def zbody(i, _):
                pltpu.sync_copy(stage_v, acc_sh.at[pl.ds(row0 + i * _ZB, _ZB)])
                return ()

            lax.fori_loop(0, rows_t // _ZB, zbody, ())

        plsc.subcore_barrier()

        base = cid * (e // nc) + sid * ep

        def body(ci, _):
            off = base + ci * _K
            pltpu.sync_copy(et_hbm.at[pl.ds(off, _K)], tv)
            pltpu.sync_copy(dst_hbm.at[pl.ds(off, _K)], dv)
            pltpu.async_copy(oh_hbm.at[tv], val_v, sem1).wait()
            pltpu.sync_copy(val_v, acc_sh.at[dv], add=True)
            return ()

        lax.fori_loop(0, nch, body, ())
        plsc.subcore_barrier()

        @pl.when(sid < zt)
        def _dump():
            def dump(i, _):
                rr = row0 + i * _ZB
                pltpu.sync_copy(acc_sh.at[pl.ds(rr, _ZB)], stage_v)
                pltpu.sync_copy(stage_v, out_hbm.at[cid, pl.ds(rr, _ZB)])
                return ()

            lax.fori_loop(0, rows_t // _ZB, dump, ())

    return deg_kernel(edge_type, dst, oh_tbl)


def _sc_agg(table, norm, edge_type, src, dst, n):
    rn, d = table.shape
    e = edge_type.shape[0]
    nc, ns = 2, 16
    ep = e // (nc * ns)
    nch = ep // _K
    zt = 10
    rows_t = n // zt
    mesh = plsc.VectorSubcoreMesh(**_MESH)

    @functools.partial(
        pl.kernel,
        out_type=jax.ShapeDtypeStruct((nc, n, d), jnp.float32),
        mesh=mesh,
        scratch_types=[
            pltpu.VMEM_SHARED((n, d), jnp.float32),
            pltpu.VMEM((_K,), jnp.int32),
            pltpu.VMEM((_K,), jnp.int32),
            pltpu.VMEM((_K,), jnp.int32),
            pltpu.VMEM((_K,), jnp.int32),
            pltpu.VMEM((_K, d), jnp.float32),
            pltpu.VMEM((_K, 128), jnp.float32),
            pltpu.VMEM((_ZB, d), jnp.float32),
            pltpu.SemaphoreType.DMA,
            pltpu.SemaphoreType.DMA,
        ],
    )
    def agg_kernel(t_hbm, nrm_hbm, et_hbm, src_hbm, dst_hbm, out_hbm,
                   acc_sh, tv, sv, dv, kdv, rows_v, coef_v, stage_v,
                   sem1, sem2):
        cid = lax.axis_index("c")
        sid = lax.axis_index("s")
        zero16 = jnp.zeros((16,), jnp.float32)
        for j in range(_ZB):
            for b_ in range(d // 16):
                stage_v[j, pl.ds(b_ * 16, 16)] = zero16
        row0 = sid * rows_t

        @pl.when(sid < zt)
        def _zero():
            def zbody(i, _):
                pltpu.sync_copy(stage_v, acc_sh.at[pl.ds(row0 + i * _ZB, _ZB)])
                return ()

            lax.fori_loop(0, rows_t // _ZB, zbody, ())

        plsc.subcore_barrier()

        base = cid * (e // nc) + sid * ep

        def body(ci, _):
            off = base + ci * _K
            pltpu.sync_copy(et_hbm.at[pl.ds(off, _K)], tv)
            pltpu.sync_copy(src_hbm.at[pl.ds(off, _K)], sv)
            pltpu.sync_copy(dst_hbm.at[pl.ds(off, _K)], dv)
            for g in range(_K // 16):
                sl = pl.ds(g * 16, 16)
                t16 = tv[sl]
                sv[sl] = t16 * n + sv[sl]
                kdv[sl] = dv[sl] * 8 + t16
            cp1 = pltpu.async_copy(t_hbm.at[sv], rows_v, sem1)
            cp2 = pltpu.async_copy(nrm_hbm.at[kdv], coef_v, sem2)
            cp1.wait()
            cp2.wait()
            for j in range(_K):
                for b_ in range(d // 16):
                    sl2 = pl.ds(b_ * 16, 16)
                    rows_v[j, sl2] = rows_v[j, sl2] * coef_v[j, sl2]
            pltpu.sync_copy(rows_v, acc_sh.at[dv], add=True)
            return ()

        lax.fori_loop(0, nch, body, ())
        plsc.subcore_barrier()

        @pl.when(sid < zt)
        def _dump():
            def dump(i, _):
                rr = row0 + i * _ZB
                pltpu.sync_copy(acc_sh.at[pl.ds(rr, _ZB)], stage_v)
                pltpu.sync_copy(stage_v, out_hbm.at[cid, pl.ds(rr, _ZB)])
                return ()

            lax.fori_loop(0, rows_t // _ZB, dump, ())

    return agg_kernel(table, norm, edge_type, src, dst)


def kernel(x, edge_index, edge_type, W1, root1, b1, W2, root2, b2):
    n, din = x.shape
    r = W1.shape[0]
    src = edge_index[0]
    dst = edge_index[1]

    oh_tbl = _onehot_table(r)
    deg2 = _sc_deg(edge_type, dst, oh_tbl, n, r)
    norm = _norm128_from_deg(deg2, n, r)

    t1 = _rel_tables(x, W1).reshape(r * n, -1)
    xr1 = _root_transform(x, root1, b1)
    p1 = _sc_agg(t1, norm, edge_type, src, dst, n)
    h = _combine(p1, xr1, relu=True)

    t2 = _rel_tables(h, W2).reshape(r * n, -1)
    xr2 = _root_transform(h, root2, b2)
    p2 = _sc_agg(t2, norm, edge_type, src, dst, n)
    return _combine(p2, xr2, relu=False)


# speedup vs baseline: 6.1838x; 1.3499x over previous
import functools
import jax
import jax.numpy as jnp
from jax import lax
from jax.experimental import pallas as pl
from jax.experimental.pallas import tpu as pltpu
from jax.experimental.pallas import tpu_sc as plsc


_BN = 400


def _rel_tables_body(x_ref, w_ref, o_ref):
    o_ref[0] = jax.lax.dot_general(
        x_ref[...], w_ref[0], (((1,), (0,)), ((), ())),
        precision=jax.lax.Precision.HIGHEST,
        preferred_element_type=jnp.float32)


def _rel_tables(x, W):
    n, din = x.shape
    r, _, dout = W.shape
    nb = n // _BN
    return pl.pallas_call(
        _rel_tables_body,
        grid=(r, nb),
        in_specs=[
            pl.BlockSpec((_BN, din), lambda ri, i: (i, 0)),
            pl.BlockSpec((1, din, dout), lambda ri, i: (ri, 0, 0)),
        ],
        out_specs=pl.BlockSpec((1, _BN, dout), lambda ri, i: (ri, i, 0)),
        out_shape=jax.ShapeDtypeStruct((r, n, dout), jnp.float32),
    )(x, W)


def _root_body(x_ref, w_ref, b_ref, o_ref):
    o_ref[...] = jax.lax.dot_general(
        x_ref[...], w_ref[...], (((1,), (0,)), ((), ())),
        precision=jax.lax.Precision.HIGHEST,
        preferred_element_type=jnp.float32) + b_ref[...]


def _root_transform(x, root, b):
    n, din = x.shape
    dout = root.shape[1]
    nb = n // _BN
    return pl.pallas_call(
        _root_body,
        grid=(nb,),
        in_specs=[
            pl.BlockSpec((_BN, din), lambda i: (i, 0)),
            pl.BlockSpec((din, dout), lambda i: (0, 0)),
            pl.BlockSpec((dout,), lambda i: (0,)),
        ],
        out_specs=pl.BlockSpec((_BN, dout), lambda i: (i, 0)),
        out_shape=jax.ShapeDtypeStruct((n, dout), jnp.float32),
    )(x, root, b)


def _norm_body(d_ref, o_ref, *, r):
    deg = d_ref[0] + d_ref[1]
    cols = []
    for ri in range(r):
        col = deg[:, 16 * ri:16 * ri + 1]
        cols.append(jnp.broadcast_to(
            1.0 / jnp.maximum(col, 1.0), (deg.shape[0], 128)))
    blk = jnp.concatenate(cols, axis=1)
    o_ref[...] = blk.reshape(-1, 128)


def _norm128_from_deg(deg2, n, r):
    nb = n // _BN
    return pl.pallas_call(
        functools.partial(_norm_body, r=r),
        grid=(nb,),
        in_specs=[pl.BlockSpec((2, _BN, 128), lambda i: (0, i, 0))],
        out_specs=pl.BlockSpec((_BN * r, 128), lambda i: (i, 0)),
        out_shape=jax.ShapeDtypeStruct((n * r, 128), jnp.float32),
    )(deg2)


def _combine_body(p_ref, xr_ref, o_ref, *, relu):
    v = p_ref[0] + p_ref[1] + xr_ref[...]
    if relu:
        v = jnp.maximum(v, 0.0)
    o_ref[...] = v


def _combine(p, xr, relu):
    n, d = xr.shape
    nb = n // _BN
    return pl.pallas_call(
        functools.partial(_combine_body, relu=relu),
        grid=(nb,),
        in_specs=[
            pl.BlockSpec((2, _BN, d), lambda i: (0, i, 0)),
            pl.BlockSpec((_BN, d), lambda i: (i, 0)),
        ],
        out_specs=pl.BlockSpec((_BN, d), lambda i: (i, 0)),
        out_shape=jax.ShapeDtypeStruct((n, d), jnp.float32),
    )(p, xr)


_K = 80
_ZB = 40

_MESH = dict(core_axis_name="c", subcore_axis_name="s")


def _onehot_body(o_ref):
    grp = jax.lax.broadcasted_iota(jnp.int32, o_ref.shape, 1) // 16
    row = jax.lax.broadcasted_iota(jnp.int32, o_ref.shape, 0)
    o_ref[...] = jnp.where(grp == row, 1.0, 0.0)


def _onehot_table(r):
    return pl.pallas_call(
        _onehot_body,
        out_shape=jax.ShapeDtypeStruct((r, 128), jnp.float32),
    )()


def _sc_deg(edge_type, dst, oh_tbl, n, r):
    e = edge_type.shape[0]
    nc, ns = 2, 16
    ep = e // (nc * ns)
    nch = ep // _K
    zt = 10
    rows_t = n // zt
    mesh = plsc.VectorSubcoreMesh(**_MESH)

    @functools.partial(
        pl.kernel,
        out_type=jax.ShapeDtypeStruct((nc, n, 128), jnp.float32),
        mesh=mesh,
        scratch_types=[
            pltpu.VMEM_SHARED((n, 128), jnp.float32),
            [[pltpu.VMEM((_K,), jnp.int32) for _ in range(2)]
             for _ in range(2)],
            [pltpu.VMEM((_K, 128), jnp.float32) for _ in range(2)],
            pltpu.VMEM((_ZB, 128), jnp.float32),
            pltpu.SemaphoreType.DMA,
            [pltpu.SemaphoreType.DMA for _ in range(2)],
            [pltpu.SemaphoreType.DMA for _ in range(2)],
        ],
    )
    def deg_kernel(et_hbm, dst_hbm, oh_hbm, out_hbm, acc_sh, idx, val,
                   stage_v, semi, semg, sems):
        cid = lax.axis_index("c")
        sid = lax.axis_index("s")
        zero16 = jnp.zeros((16,), jnp.float32)
        for j in range(_ZB):
            for b_ in range(8):
                stage_v[j, pl.ds(b_ * 16, 16)] = zero16
        row0 = sid * rows_t

        @pl.when(sid < zt)
        def _zero():
            def zbody(i, _):
                pltpu.sync_copy(stage_v, acc_sh.at[pl.ds(row0 + i * _ZB, _ZB)])
                return ()

            lax.fori_loop(0, rows_t // _ZB, zbody, ())

        plsc.subcore_barrier()

        base = cid * (e // nc) + sid * ep

        def load_idx(off, p):
            tv, dv = idx[p]
            return [pltpu.async_copy(et_hbm.at[pl.ds(off, _K)], tv, semi),
                    pltpu.async_copy(dst_hbm.at[pl.ds(off, _K)], dv, semi)]

        def pair(i, _):
            off0 = base + (2 * i) * _K
            la = load_idx(off0, 0)
            lb = load_idx(off0 + _K, 1)
            for c in la:
                c.wait()
            ga = pltpu.async_copy(oh_hbm.at[idx[0][0]], val[0], semg[0])
            for c in lb:
                c.wait()
            gb = pltpu.async_copy(oh_hbm.at[idx[1][0]], val[1], semg[1])
            ga.wait()
            sa = pltpu.async_copy(val[0], acc_sh.at[idx[0][1]], sems[0],
                                  add=True)
            gb.wait()
            sb = pltpu.async_copy(val[1], acc_sh.at[idx[1][1]], sems[1],
                                  add=True)
            sa.wait()
            sb.wait()
            return ()

        lax.fori_loop(0, nch // 2, pair, ())
        if nch % 2:
            off0 = base + (nch - 1) * _K
            for c in load_idx(off0, 0):
                c.wait()
            pltpu.async_copy(oh_hbm.at[idx[0][0]], val[0], semg[0]).wait()
            pltpu.async_copy(val[0], acc_sh.at[idx[0][1]], sems[0],
                             add=True).wait()
        plsc.subcore_barrier()

        @pl.when(sid < zt)
        def _dump():
            def dump(i, _):
                rr = row0 + i * _ZB
                pltpu.sync_copy(acc_sh.at[pl.ds(rr, _ZB)], stage_v)
                pltpu.sync_copy(stage_v, out_hbm.at[cid, pl.ds(rr, _ZB)])
                return ()

            lax.fori_loop(0, rows_t // _ZB, dump, ())

    return deg_kernel(edge_type, dst, oh_tbl)


def _sc_agg(table, norm, edge_type, src, dst, n):
    rn, d = table.shape
    e = edge_type.shape[0]
    nc, ns = 2, 16
    ep = e // (nc * ns)
    nch = ep // _K
    zt = 10
    rows_t = n // zt
    mesh = plsc.VectorSubcoreMesh(**_MESH)

    @functools.partial(
        pl.kernel,
        out_type=jax.ShapeDtypeStruct((nc, n, d), jnp.float32),
        mesh=mesh,
        scratch_types=[
            pltpu.VMEM_SHARED((n, d), jnp.float32),
            [[pltpu.VMEM((_K,), jnp.int32) for _ in range(4)]
             for _ in range(2)],
            [pltpu.VMEM((_K, d), jnp.float32) for _ in range(2)],
            [pltpu.VMEM((_K, 128), jnp.float32) for _ in range(2)],
            pltpu.VMEM((_ZB, d), jnp.float32),
            pltpu.SemaphoreType.DMA,
            [pltpu.SemaphoreType.DMA for _ in range(2)],
            [pltpu.SemaphoreType.DMA for _ in range(2)],
        ],
    )
    def agg_kernel(t_hbm, nrm_hbm, et_hbm, src_hbm, dst_hbm, out_hbm,
                   acc_sh, idx, rows, coef, stage_v, semi, semg, sems):
        cid = lax.axis_index("c")
        sid = lax.axis_index("s")
        zero16 = jnp.zeros((16,), jnp.float32)
        for j in range(_ZB):
            for b_ in range(d // 16):
                stage_v[j, pl.ds(b_ * 16, 16)] = zero16
        row0 = sid * rows_t

        @pl.when(sid < zt)
        def _zero():
            def zbody(i, _):
                pltpu.sync_copy(stage_v, acc_sh.at[pl.ds(row0 + i * _ZB, _ZB)])
                return ()

            lax.fori_loop(0, rows_t // _ZB, zbody, ())

        plsc.subcore_barrier()

        base = cid * (e // nc) + sid * ep

        def load_idx(off, p):
            tv, sv, dv, _ = idx[p]
            return [pltpu.async_copy(et_hbm.at[pl.ds(off, _K)], tv, semi),
                    pltpu.async_copy(src_hbm.at[pl.ds(off, _K)], sv, semi),
                    pltpu.async_copy(dst_hbm.at[pl.ds(off, _K)], dv, semi)]

        def keys(p):
            tv, sv, dv, kdv = idx[p]
            for g in range(_K // 16):
                sl = pl.ds(g * 16, 16)
                t16 = tv[sl]
                sv[sl] = t16 * n + sv[sl]
                kdv[sl] = dv[sl] * 8 + t16

        def gathers(p):
            _, sv, _, kdv = idx[p]
            return [pltpu.async_copy(t_hbm.at[sv], rows[p], semg[p]),
                    pltpu.async_copy(nrm_hbm.at[kdv], coef[p], semg[p])]

        def scale(p):
            def srow(j, _):
                for b_ in range(d // 16):
                    sl2 = pl.ds(b_ * 16, 16)
                    rows[p][j, sl2] = rows[p][j, sl2] * coef[p][j, sl2]
                return ()

            lax.fori_loop(0, _K, srow, ())

        def scatter(p):
            return pltpu.async_copy(rows[p], acc_sh.at[idx[p][2]], sems[p],
                                    add=True)

        def pair(i, _):
            off0 = base + (2 * i) * _K
            la = load_idx(off0, 0)
            lb = load_idx(off0 + _K, 1)
            for c in la:
                c.wait()
            keys(0)
            ga = gathers(0)
            for c in lb:
                c.wait()
            keys(1)
            gb = gathers(1)
            for c in ga:
                c.wait()
            scale(0)
            sa = scatter(0)
            for c in gb:
                c.wait()
            scale(1)
            sb = scatter(1)
            sa.wait()
            sb.wait()
            return ()

        lax.fori_loop(0, nch // 2, pair, ())
        if nch % 2:
            off0 = base + (nch - 1) * _K
            for c in load_idx(off0, 0):
                c.wait()
            keys(0)
            for c in gathers(0):
                c.wait()
            scale(0)
            scatter(0).wait()
        plsc.subcore_barrier()

        @pl.when(sid < zt)
        def _dump():
            def dump(i, _):
                rr = row0 + i * _ZB
                pltpu.sync_copy(acc_sh.at[pl.ds(rr, _ZB)], stage_v)
                pltpu.sync_copy(stage_v, out_hbm.at[cid, pl.ds(rr, _ZB)])
                return ()

            lax.fori_loop(0, rows_t // _ZB, dump, ())

    return agg_kernel(table, norm, edge_type, src, dst)


def kernel(x, edge_index, edge_type, W1, root1, b1, W2, root2, b2):
    n, din = x.shape
    r = W1.shape[0]
    src = edge_index[0]
    dst = edge_index[1]

    oh_tbl = _onehot_table(r)
    deg2 = _sc_deg(edge_type, dst, oh_tbl, n, r)
    norm = _norm128_from_deg(deg2, n, r)

    t1 = _rel_tables(x, W1).reshape(r * n, -1)
    xr1 = _root_transform(x, root1, b1)
    p1 = _sc_agg(t1, norm, edge_type, src, dst, n)
    h = _combine(p1, xr1, relu=True)

    t2 = _rel_tables(h, W2).reshape(r * n, -1)
    xr2 = _root_transform(h, root2, b2)
    p2 = _sc_agg(t2, norm, edge_type, src, dst, n)
    return _combine(p2, xr2, relu=False)


# speedup vs baseline: 13.0648x; 2.1128x over previous
import functools
import jax
import jax.numpy as jnp
from jax import lax
from jax.experimental import pallas as pl
from jax.experimental.pallas import tpu as pltpu
from jax.experimental.pallas import tpu_sc as plsc


_BN = 400


def _rel_tables_body(x_ref, w_ref, o_ref):
    o_ref[0] = jax.lax.dot_general(
        x_ref[...], w_ref[0], (((1,), (0,)), ((), ())),
        precision=jax.lax.Precision.HIGHEST,
        preferred_element_type=jnp.float32)


def _rel_tables(x, W):
    n, din = x.shape
    r, _, dout = W.shape
    nb = n // _BN
    return pl.pallas_call(
        _rel_tables_body,
        grid=(r, nb),
        in_specs=[
            pl.BlockSpec((_BN, din), lambda ri, i: (i, 0)),
            pl.BlockSpec((1, din, dout), lambda ri, i: (ri, 0, 0)),
        ],
        out_specs=pl.BlockSpec((1, _BN, dout), lambda ri, i: (ri, i, 0)),
        out_shape=jax.ShapeDtypeStruct((r, n, dout), jnp.float32),
    )(x, W)


def _root_body(x_ref, w_ref, b_ref, o_ref):
    o_ref[...] = jax.lax.dot_general(
        x_ref[...], w_ref[...], (((1,), (0,)), ((), ())),
        precision=jax.lax.Precision.HIGHEST,
        preferred_element_type=jnp.float32) + b_ref[...]


def _root_transform(x, root, b):
    n, din = x.shape
    dout = root.shape[1]
    nb = n // _BN
    return pl.pallas_call(
        _root_body,
        grid=(nb,),
        in_specs=[
            pl.BlockSpec((_BN, din), lambda i: (i, 0)),
            pl.BlockSpec((din, dout), lambda i: (0, 0)),
            pl.BlockSpec((dout,), lambda i: (0,)),
        ],
        out_specs=pl.BlockSpec((_BN, dout), lambda i: (i, 0)),
        out_shape=jax.ShapeDtypeStruct((n, dout), jnp.float32),
    )(x, root, b)


def _norm_body(d_ref, o_ref, *, r):
    deg = d_ref[0] + d_ref[1]
    cols = []
    for ri in range(r):
        col = deg[:, 16 * ri:16 * ri + 1]
        cols.append(jnp.broadcast_to(
            1.0 / jnp.maximum(col, 1.0), (deg.shape[0], 128)))
    blk = jnp.concatenate(cols, axis=1)
    o_ref[...] = blk.reshape(-1, 128)


def _norm128_from_deg(deg2, n, r):
    nb = n // _BN
    return pl.pallas_call(
        functools.partial(_norm_body, r=r),
        grid=(nb,),
        in_specs=[pl.BlockSpec((2, _BN, 128), lambda i: (0, i, 0))],
        out_specs=pl.BlockSpec((_BN * r, 128), lambda i: (i, 0)),
        out_shape=jax.ShapeDtypeStruct((n * r, 128), jnp.float32),
    )(deg2)


def _combine_body(p_ref, xr_ref, o_ref, *, relu):
    v = p_ref[0] + p_ref[1] + xr_ref[...]
    if relu:
        v = jnp.maximum(v, 0.0)
    o_ref[...] = v


def _combine(p, xr, relu):
    n, d = xr.shape
    nb = n // _BN
    return pl.pallas_call(
        functools.partial(_combine_body, relu=relu),
        grid=(nb,),
        in_specs=[
            pl.BlockSpec((2, _BN, d), lambda i: (0, i, 0)),
            pl.BlockSpec((_BN, d), lambda i: (i, 0)),
        ],
        out_specs=pl.BlockSpec((_BN, d), lambda i: (i, 0)),
        out_shape=jax.ShapeDtypeStruct((n, d), jnp.float32),
    )(p, xr)


_K = 80
_ZB = 40

_MESH = dict(core_axis_name="c", subcore_axis_name="s")


def _onehot_body(o_ref, *, r):
    grp = jax.lax.broadcasted_iota(jnp.int32, o_ref.shape, 1) // 16
    row = jax.lax.broadcasted_iota(jnp.int32, o_ref.shape, 0) % r
    o_ref[...] = jnp.where(grp == row, 1.0, 0.0)


def _onehot_table(r, copies):
    return pl.pallas_call(
        functools.partial(_onehot_body, r=r),
        out_shape=jax.ShapeDtypeStruct((copies * r, 128), jnp.float32),
    )()


def _sc_deg(edge_type, dst, oh_tbl, n, r):
    e = edge_type.shape[0]
    nc, ns = 2, 16
    ep = e // (nc * ns)
    nch = ep // _K
    zt = 10
    rows_t = n // zt
    mesh = plsc.VectorSubcoreMesh(**_MESH)

    @functools.partial(
        pl.kernel,
        out_type=jax.ShapeDtypeStruct((nc, n, 128), jnp.float32),
        mesh=mesh,
        scratch_types=[
            pltpu.VMEM_SHARED((n, 128), jnp.float32),
            [[pltpu.VMEM((_K,), jnp.int32) for _ in range(2)]
             for _ in range(2)],
            [pltpu.VMEM((_K, 128), jnp.float32) for _ in range(2)],
            pltpu.VMEM((_ZB, 128), jnp.float32),
            pltpu.SemaphoreType.DMA,
            [pltpu.SemaphoreType.DMA for _ in range(2)],
            [pltpu.SemaphoreType.DMA for _ in range(2)],
        ],
    )
    def deg_kernel(et_hbm, dst_hbm, oh_hbm, out_hbm, acc_sh, idx, val,
                   stage_v, semi, semg, sems):
        cid = lax.axis_index("c")
        sid = lax.axis_index("s")
        zero16 = jnp.zeros((16,), jnp.float32)
        for j in range(_ZB):
            for b_ in range(8):
                stage_v[j, pl.ds(b_ * 16, 16)] = zero16
        row0 = sid * rows_t

        @pl.when(sid < zt)
        def _zero():
            def zbody(i, _):
                pltpu.sync_copy(stage_v, acc_sh.at[pl.ds(row0 + i * _ZB, _ZB)])
                return ()

            lax.fori_loop(0, rows_t // _ZB, zbody, ())

        plsc.subcore_barrier()

        base = cid * (e // nc) + sid * ep

        wid = sid * nc + cid

        def load_idx(off, p):
            tv, dv = idx[p]
            return [pltpu.async_copy(et_hbm.at[pl.ds(off, _K)], tv, semi),
                    pltpu.async_copy(dst_hbm.at[pl.ds(off, _K)], dv, semi)]

        def privatize(p):
            tv = idx[p][0]
            for g in range(_K // 16):
                sl = pl.ds(g * 16, 16)
                tv[sl] = tv[sl] + wid * r

        def pair(i, _):
            off0 = base + (2 * i) * _K
            la = load_idx(off0, 0)
            lb = load_idx(off0 + _K, 1)
            for c in la:
                c.wait()
            privatize(0)
            ga = pltpu.async_copy(oh_hbm.at[idx[0][0]], val[0], semg[0])
            for c in lb:
                c.wait()
            privatize(1)
            gb = pltpu.async_copy(oh_hbm.at[idx[1][0]], val[1], semg[1])
            ga.wait()
            sa = pltpu.async_copy(val[0], acc_sh.at[idx[0][1]], sems[0],
                                  add=True)
            gb.wait()
            sb = pltpu.async_copy(val[1], acc_sh.at[idx[1][1]], sems[1],
                                  add=True)
            sa.wait()
            sb.wait()
            return ()

        lax.fori_loop(0, nch // 2, pair, ())
        if nch % 2:
            off0 = base + (nch - 1) * _K
            for c in load_idx(off0, 0):
                c.wait()
            privatize(0)
            pltpu.async_copy(oh_hbm.at[idx[0][0]], val[0], semg[0]).wait()
            pltpu.async_copy(val[0], acc_sh.at[idx[0][1]], sems[0],
                             add=True).wait()
        plsc.subcore_barrier()

        @pl.when(sid < zt)
        def _dump():
            def dump(i, _):
                rr = row0 + i * _ZB
                pltpu.sync_copy(acc_sh.at[pl.ds(rr, _ZB)], stage_v)
                pltpu.sync_copy(stage_v, out_hbm.at[cid, pl.ds(rr, _ZB)])
                return ()

            lax.fori_loop(0, rows_t // _ZB, dump, ())

    return deg_kernel(edge_type, dst, oh_tbl)


def _sc_agg(table, norm, edge_type, src, dst, n):
    rn, d = table.shape
    e = edge_type.shape[0]
    nc, ns = 2, 16
    ep = e // (nc * ns)
    nch = ep // _K
    zt = 10
    rows_t = n // zt
    mesh = plsc.VectorSubcoreMesh(**_MESH)

    @functools.partial(
        pl.kernel,
        out_type=jax.ShapeDtypeStruct((nc, n, d), jnp.float32),
        mesh=mesh,
        scratch_types=[
            pltpu.VMEM_SHARED((n, d), jnp.float32),
            [[pltpu.VMEM((_K,), jnp.int32) for _ in range(4)]
             for _ in range(2)],
            [pltpu.VMEM((_K, d), jnp.float32) for _ in range(2)],
            [pltpu.VMEM((_K, 128), jnp.float32) for _ in range(2)],
            pltpu.VMEM((_ZB, d), jnp.float32),
            pltpu.SemaphoreType.DMA,
            [pltpu.SemaphoreType.DMA for _ in range(2)],
            [pltpu.SemaphoreType.DMA for _ in range(2)],
        ],
    )
    def agg_kernel(t_hbm, nrm_hbm, et_hbm, src_hbm, dst_hbm, out_hbm,
                   acc_sh, idx, rows, coef, stage_v, semi, semg, sems):
        cid = lax.axis_index("c")
        sid = lax.axis_index("s")
        zero16 = jnp.zeros((16,), jnp.float32)
        for j in range(_ZB):
            for b_ in range(d // 16):
                stage_v[j, pl.ds(b_ * 16, 16)] = zero16
        row0 = sid * rows_t

        @pl.when(sid < zt)
        def _zero():
            def zbody(i, _):
                pltpu.sync_copy(stage_v, acc_sh.at[pl.ds(row0 + i * _ZB, _ZB)])
                return ()

            lax.fori_loop(0, rows_t // _ZB, zbody, ())

        plsc.subcore_barrier()

        base = cid * (e // nc) + sid * ep

        def load_idx(off, p):
            tv, sv, dv, _ = idx[p]
            return [pltpu.async_copy(et_hbm.at[pl.ds(off, _K)], tv, semi),
                    pltpu.async_copy(src_hbm.at[pl.ds(off, _K)], sv, semi),
                    pltpu.async_copy(dst_hbm.at[pl.ds(off, _K)], dv, semi)]

        def keys(p):
            tv, sv, dv, kdv = idx[p]
            for g in range(_K // 16):
                sl = pl.ds(g * 16, 16)
                t16 = tv[sl]
                sv[sl] = t16 * n + sv[sl]
                kdv[sl] = dv[sl] * 8 + t16

        def gathers(p):
            _, sv, _, kdv = idx[p]
            return [pltpu.async_copy(t_hbm.at[sv], rows[p], semg[p]),
                    pltpu.async_copy(nrm_hbm.at[kdv], coef[p], semg[p])]

        def scale(p):
            def srow(j, _):
                for b_ in range(d // 16):
                    sl2 = pl.ds(b_ * 16, 16)
                    rows[p][j, sl2] = rows[p][j, sl2] * coef[p][j, sl2]
                return ()

            lax.fori_loop(0, _K, srow, ())

        def scatter(p):
            return pltpu.async_copy(rows[p], acc_sh.at[idx[p][2]], sems[p],
                                    add=True)

        def pair(i, _):
            off0 = base + (2 * i) * _K
            la = load_idx(off0, 0)
            lb = load_idx(off0 + _K, 1)
            for c in la:
                c.wait()
            keys(0)
            ga = gathers(0)
            for c in lb:
                c.wait()
            keys(1)
            gb = gathers(1)
            for c in ga:
                c.wait()
            scale(0)
            sa = scatter(0)
            for c in gb:
                c.wait()
            scale(1)
            sb = scatter(1)
            sa.wait()
            sb.wait()
            return ()

        lax.fori_loop(0, nch // 2, pair, ())
        if nch % 2:
            off0 = base + (nch - 1) * _K
            for c in load_idx(off0, 0):
                c.wait()
            keys(0)
            for c in gathers(0):
                c.wait()
            scale(0)
            scatter(0).wait()
        plsc.subcore_barrier()

        @pl.when(sid < zt)
        def _dump():
            def dump(i, _):
                rr = row0 + i * _ZB
                pltpu.sync_copy(acc_sh.at[pl.ds(rr, _ZB)], stage_v)
                pltpu.sync_copy(stage_v, out_hbm.at[cid, pl.ds(rr, _ZB)])
                return ()

            lax.fori_loop(0, rows_t // _ZB, dump, ())

    return agg_kernel(table, norm, edge_type, src, dst)


def kernel(x, edge_index, edge_type, W1, root1, b1, W2, root2, b2):
    n, din = x.shape
    r = W1.shape[0]
    src = edge_index[0]
    dst = edge_index[1]

    oh_tbl = _onehot_table(r, 32)
    deg2 = _sc_deg(edge_type, dst, oh_tbl, n, r)
    norm = _norm128_from_deg(deg2, n, r)

    t1 = _rel_tables(x, W1).reshape(r * n, -1)
    xr1 = _root_transform(x, root1, b1)
    p1 = _sc_agg(t1, norm, edge_type, src, dst, n)
    h = _combine(p1, xr1, relu=True)

    t2 = _rel_tables(h, W2).reshape(r * n, -1)
    xr2 = _root_transform(h, root2, b2)
    p2 = _sc_agg(t2, norm, edge_type, src, dst, n)
    return _combine(p2, xr2, relu=False)
